# baseline (speedup 1.0000x reference)
import functools

import jax
import jax.numpy as jnp
import numpy as np
from jax import lax
from jax.experimental import pallas as pl
from jax.experimental.pallas import tpu as pltpu
from jax.experimental.pallas import tpu_sc as plsc

_info = plsc.get_sparse_core_info()
_NC, _NS = _info.num_cores, _info.num_subcores
_NW = _NC * _NS


@functools.lru_cache(maxsize=None)
def _build_sc_call(batch, h, w, dim):
    n_rows = 2 * h * w
    assert n_rows % _NW == 0
    rpw = n_rows // _NW

    r = np.arange(n_rows)
    y, xh = r // (2 * w), r % (2 * w)
    x, half = xh // 2, xh % 2
    idx_np = np.where(half == 0, x, w + y).astype(np.int32)

    mesh = plsc.VectorSubcoreMesh(core_axis_name="c", subcore_axis_name="s")

    @functools.partial(
        pl.kernel,
        mesh=mesh,
        out_type=jax.ShapeDtypeStruct((batch, n_rows, dim), jnp.float32),
        scratch_types=[
            pltpu.VMEM((rpw,), jnp.int32),
            pltpu.VMEM((rpw, dim), jnp.float32),
            pltpu.SemaphoreType.DMA,
        ],
    )
    def sc_kernel(table_hbm, idx_hbm, out_hbm, idx_v, rows_v, sem):
        wid = lax.axis_index("s") * _NC + lax.axis_index("c")
        base = wid * rpw
        pltpu.sync_copy(idx_hbm.at[pl.ds(base, rpw)], idx_v)
        pltpu.async_copy(table_hbm.at[idx_v], rows_v, sem).wait()
        copies = [
            pltpu.async_copy(rows_v, out_hbm.at[b, pl.ds(base, rpw)], sem)
            for b in range(batch)
        ]
        for c in copies:
            c.wait()

    idx = jnp.asarray(idx_np)

    def call(table):
        return sc_kernel(table, idx)

    return call


def kernel(inputs, row_embed, col_embed):
    batch, h, w, _ = inputs.shape
    dim = col_embed.shape[1]
    table = jnp.concatenate([col_embed[:w], row_embed[:h]], axis=0)
    out = _build_sc_call(batch, h, w, dim)(table)
    return out.reshape(batch, h, w, 2 * dim)


# speedup vs baseline: 5.3609x; 5.3609x over previous
import functools

import jax
import jax.numpy as jnp
from jax.experimental import pallas as pl
from jax.experimental.pallas import tpu as pltpu


@functools.lru_cache(maxsize=None)
def _build_call(batch, h, w, dim):
    def body(col_ref, row_ref, out_ref, img, sem):
        col = col_ref[...]
        row = row_ref[...]
        left = jnp.broadcast_to(col[None, :, :], (h, w, dim))
        right = jnp.broadcast_to(row[:, None, :], (h, w, dim))
        img[...] = jnp.concatenate([left, right], axis=-1)
        copies = [
            pltpu.make_async_copy(img, out_ref.at[b], sem) for b in range(batch)
        ]
        for c in copies:
            c.start()
        for c in copies:
            c.wait()

    return pl.pallas_call(
        body,
        out_shape=jax.ShapeDtypeStruct((batch, h, w, 2 * dim), jnp.float32),
        in_specs=[
            pl.BlockSpec(memory_space=pltpu.VMEM),
            pl.BlockSpec(memory_space=pltpu.VMEM),
        ],
        out_specs=pl.BlockSpec(memory_space=pl.ANY),
        scratch_shapes=[
            pltpu.VMEM((h, w, 2 * dim), jnp.float32),
            pltpu.SemaphoreType.DMA,
        ],
    )


def kernel(inputs, row_embed, col_embed):
    batch, h, w, _ = inputs.shape
    dim = col_embed.shape[1]
    return _build_call(batch, h, w, dim)(col_embed[:w], row_embed[:h])


# speedup vs baseline: 5.3716x; 1.0020x over previous
import functools

import jax
import jax.numpy as jnp
from jax.experimental import pallas as pl
from jax.experimental.pallas import tpu as pltpu


@functools.lru_cache(maxsize=None)
def _build_call(batch, h, w, dim):
    def body(col_ref, row_ref, out_ref, img, sem):
        col = col_ref[...]
        row = row_ref[...]
        left = jnp.broadcast_to(col[None, :, :], (h, w, dim))
        right = jnp.broadcast_to(row[:, None, :], (h, w, dim))
        img[...] = jnp.concatenate([left, right], axis=-1)
        nchunk = 4
        ch = h // nchunk
        copies = [
            pltpu.make_async_copy(
                img.at[pl.ds(c * ch, ch)],
                out_ref.at[b, pl.ds(c * ch, ch)],
                sem,
            )
            for b in range(batch)
            for c in range(nchunk)
        ]
        for c in copies:
            c.start()
        for c in copies:
            c.wait()

    return pl.pallas_call(
        body,
        out_shape=jax.ShapeDtypeStruct((batch, h, w, 2 * dim), jnp.float32),
        in_specs=[
            pl.BlockSpec(memory_space=pltpu.VMEM),
            pl.BlockSpec(memory_space=pltpu.VMEM),
        ],
        out_specs=pl.BlockSpec(memory_space=pl.ANY),
        scratch_shapes=[
            pltpu.VMEM((h, w, 2 * dim), jnp.float32),
            pltpu.SemaphoreType.DMA,
        ],
    )


def kernel(inputs, row_embed, col_embed):
    batch, h, w, _ = inputs.shape
    dim = col_embed.shape[1]
    return _build_call(batch, h, w, dim)(col_embed[:w], row_embed[:h])
